# baseline (speedup 1.0000x reference)
import jax
import jax.numpy as jnp
from jax import lax
from jax.experimental import pallas as pl
from jax.experimental.pallas import tpu as pltpu
from jax.experimental.pallas import tpu_sc as plsc

N = 10000
E = 320000
B = 4
IN = 128
H = 128
OUT = 16

NCORES = 2
NTILES = 16
BPC = B // NCORES
PADN = 10240
RPT = PADN // NTILES
CHUNK = 128
EPAD = 327680
NCHUNKS = EPAD // CHUNK
CPT = NCHUNKS // NTILES
CPT_DEG = NCHUNKS // (2 * NTILES)
NBUF = 4
MBLK = 400
TROWS = 40960
LBLK = 512
CVT_RPT = TROWS // 32
CVT_CH = 64
CVT_NCH = CVT_RPT // CVT_CH

_SC_MESH = dict(core_axis_name="c", subcore_axis_name="s")


def _lstm_gcn1_body(x_ref, wih_ref, bias_ref, w1_ref, dinv_ref, out_ref):
    xg = x_ref[...] @ wih_ref[...] + bias_ref[...]
    i_g = xg[:, 0:H]
    g_g = xg[:, 2 * H:3 * H]
    o_g = xg[:, 3 * H:4 * H]
    c = jax.nn.sigmoid(i_g) * jnp.tanh(g_g)
    h = jax.nn.sigmoid(o_g) * jnp.tanh(c)
    h1 = (h @ w1_ref[...]) * dinv_ref[0]
    out_ref[...] = h1


def _combine_gcn2_body(s_ref, ht_ref, dinv_ref, b1_ref, w2_ref, out_ref):
    dinv = dinv_ref[0]
    s = s_ref[0].astype(jnp.float32)
    ht = ht_ref[...].astype(jnp.float32)
    x2 = jax.nn.relu(dinv * (s + ht) + b1_ref[...])
    h2 = (x2 @ w2_ref[...]) * dinv
    out_ref[...] = h2


def _final_body(s_ref, ht_ref, dinv_ref, b2_ref, w3_ref, b3_ref, w4_ref,
                b4_ref, out_ref):
    dinv = dinv_ref[0]
    s = s_ref[0].astype(jnp.float32)
    ht = ht_ref[...].astype(jnp.float32)
    x3 = jax.nn.relu(dinv * (s + ht) + b2_ref[...])
    y = jax.nn.relu(x3 @ w3_ref[...] + b3_ref[...])
    out_ref[...] = y @ w4_ref[...] + b4_ref[...]


def _dinv_body(deg_ref, out_ref):
    out_ref[...] = lax.rsqrt(deg_ref[...] + 1.0)


def _deg_body(dst2d, out, didx_v, ones_v, acc_sh, sem):
    core = lax.axis_index("c")
    sub = lax.axis_index("s")
    zeros16 = jnp.zeros((16,), jnp.float32)
    ones16 = jnp.ones((16,), jnp.float32)
    for k in range(CHUNK // 16):
        ones_v[pl.ds(k * 16, 16)] = ones16

    def zrow(zv):
        for k in range(RPT // 16):
            zv[pl.ds(k * 16, 16)] = zeros16
        pltpu.sync_copy(zv, acc_sh.at[pl.ds(sub * RPT, RPT)])
    pl.run_scoped(zrow, pltpu.VMEM((RPT,), jnp.float32))
    plsc.subcore_barrier()
    base = (core * NTILES + sub) * CPT_DEG
    pltpu.sync_copy(dst2d.at[pl.ds(base, CPT_DEG)], didx_v)

    def body(j, _):
        pltpu.sync_copy(ones_v, acc_sh.at[didx_v.at[j]], add=True)
        return 0

    lax.fori_loop(0, CPT_DEG, body, 0)
    plsc.subcore_barrier()
    pltpu.sync_copy(acc_sh.at[pl.ds(sub * RPT, RPT)],
                    out.at[core, pl.ds(sub * RPT, RPT)])


def _cvt_body(tablef, tbl16, fbuf, bbuf):
    core = lax.axis_index("c")
    sub = lax.axis_index("s")
    gid = core * NTILES + sub
    r0 = gid * CVT_RPT
    iota2 = lax.iota(jnp.int32, 16) * 2
    idx_eo = [(iota2 + c * 32, iota2 + c * 32 + 1) for c in range(4)]

    def chunk(k, _):
        base = r0 + k * CVT_CH
        pltpu.sync_copy(tablef.at[pl.ds(base, CVT_CH)], fbuf)

        def row(r, _):
            ridx = jnp.full((16,), r, jnp.int32)
            for c in range(4):
                a = plsc.load_gather(fbuf, [ridx, idx_eo[c][0]])
                b = plsc.load_gather(fbuf, [ridx, idx_eo[c][1]])
                packed = plsc.pack(a, b, format=plsc.PackFormat.INTERLEAVED)
                bbuf[r, pl.ds(c * 32, 32)] = packed
            return 0

        lax.fori_loop(0, CVT_CH, row, 0)
        pltpu.sync_copy(bbuf, tbl16.at[pl.ds(base, CVT_CH)])
        return 0

    lax.fori_loop(0, CVT_NCH, chunk, 0)


def _sc_convert(tablef):
    kfn = pl.kernel(
        _cvt_body,
        out_type=jax.ShapeDtypeStruct((TROWS, H), jnp.bfloat16),
        mesh=plsc.VectorSubcoreMesh(**_SC_MESH),
        compiler_params=pltpu.CompilerParams(use_tc_tiling_on_sc=False,
                                             needs_layout_passes=False),
        scratch_types=[
            pltpu.VMEM((CVT_CH, H), jnp.float32),
            pltpu.VMEM((CVT_CH, H), jnp.bfloat16),
        ],
    )
    return kfn(tablef)


def _spmm_body(table, srcb, dst2d, zeros_hbm, out, sidx_v, didx_v, rows_v,
               acc_sh, gsem, ssem):
    core = lax.axis_index("c")
    sub = lax.axis_index("s")
    row0 = sub * RPT
    cbase = sub * CPT
    pltpu.sync_copy(dst2d.at[pl.ds(cbase, CPT)], didx_v)

    for bl in range(BPC):
        batch = core * BPC + bl
        pltpu.sync_copy(zeros_hbm.at[pl.ds(row0, RPT)],
                        acc_sh.at[pl.ds(row0, RPT)])
        pltpu.sync_copy(srcb.at[batch, pl.ds(cbase, CPT)], sidx_v)
        plsc.subcore_barrier()
        for k in range(NBUF):
            pltpu.async_copy(table.at[sidx_v.at[k]], rows_v.at[k],
                             gsem.at[k])

        def body(j, _):
            s = lax.rem(j, NBUF)
            pltpu.make_async_copy(table.at[sidx_v.at[j]], rows_v.at[s],
                                  gsem.at[s]).wait()
            pltpu.async_copy(rows_v.at[s], acc_sh.at[didx_v.at[j]],
                             ssem.at[s], add=True)
            jn = j + NBUF

            @pl.when(jn < CPT)
            def _():
                pltpu.make_async_copy(rows_v.at[s], acc_sh.at[didx_v.at[j]],
                                      ssem.at[s]).wait()
                pltpu.async_copy(table.at[sidx_v.at[jn]], rows_v.at[s],
                                 gsem.at[s])

            return 0

        lax.fori_loop(0, CPT, body, 0)
        for k in range(NBUF):
            pltpu.make_async_copy(rows_v.at[k], acc_sh.at[didx_v.at[0]],
                                  ssem.at[k]).wait()
        plsc.subcore_barrier()
        pltpu.sync_copy(acc_sh.at[pl.ds(row0, RPT)],
                        out.at[batch, pl.ds(row0, RPT)])


def _sc_degree(dst2d):
    kfn = pl.kernel(
        _deg_body,
        out_type=jax.ShapeDtypeStruct((NCORES, PADN), jnp.float32),
        mesh=plsc.VectorSubcoreMesh(**_SC_MESH),
        scratch_types=[
            pltpu.VMEM((CPT_DEG, CHUNK), jnp.int32),
            pltpu.VMEM((CHUNK,), jnp.float32),
            pltpu.VMEM_SHARED((PADN,), jnp.float32),
            pltpu.SemaphoreType.DMA,
        ],
    )
    return kfn(dst2d)


def _sc_spmm(table, srcb, dst2d, zeros_hbm):
    kfn = pl.kernel(
        _spmm_body,
        out_type=jax.ShapeDtypeStruct((B, PADN, H), jnp.bfloat16),
        mesh=plsc.VectorSubcoreMesh(**_SC_MESH),
        compiler_params=pltpu.CompilerParams(use_tc_tiling_on_sc=False),
        scratch_types=[
            pltpu.VMEM((CPT, CHUNK), jnp.int32),
            pltpu.VMEM((CPT, CHUNK), jnp.int32),
            pltpu.VMEM((NBUF, CHUNK, H), jnp.bfloat16),
            pltpu.VMEM_SHARED((PADN, H), jnp.bfloat16),
            pltpu.SemaphoreType.DMA((NBUF,)),
            pltpu.SemaphoreType.DMA((NBUF,)),
        ],
    )
    return kfn(table, srcb, dst2d, zeros_hbm)


def kernel(x, edge_index, w_ih, w_hh, b_ih, b_hh, gcn1_w, gcn1_b, gcn2_w,
           gcn2_b, fc1_w, fc1_b, fc2_w, fc2_b):
    f32 = jnp.float32
    bf16 = jnp.bfloat16
    i32 = jnp.int32
    xl = x.reshape(B * N, IN)
    src = edge_index[0]
    dst = edge_index[1]

    pad = EPAD - E
    pad_src = (jnp.arange(pad, dtype=i32) % N)
    pad_dst = N + (jnp.arange(pad, dtype=i32) % (PADN - N))
    src2d = jnp.concatenate([src, pad_src]).reshape(NCHUNKS, CHUNK)
    dst2d = jnp.concatenate([dst, pad_dst]).reshape(NCHUNKS, CHUNK)
    srcb = src2d[None] + (jnp.arange(B, dtype=i32) * N)[:, None, None]
    zeros_hbm = jnp.zeros((PADN, H), bf16)

    deg_parts = _sc_degree(dst2d)
    dinv = pl.pallas_call(
        _dinv_body,
        out_shape=jax.ShapeDtypeStruct((PADN // 128, 128), f32),
    )(
        (deg_parts[0] + deg_parts[1]).reshape(PADN // 128, 128))
    dinv_n = dinv.reshape(PADN)[:N]
    dinv3d = dinv_n.reshape(N // MBLK, MBLK, 1)
    dinv4 = jnp.concatenate(
        [jnp.tile(dinv_n, B), jnp.ones((TROWS - B * N,), f32)]
    ).reshape(TROWS // LBLK, LBLK, 1)
    xlp = jnp.concatenate([xl, jnp.zeros((TROWS - B * N, IN), f32)])

    wihT = w_ih.T
    bias = (b_ih + b_hh).reshape(1, 4 * H)
    nblk = TROWS // LBLK
    ht1 = pl.pallas_call(
        _lstm_gcn1_body,
        grid=(nblk,),
        in_specs=[
            pl.BlockSpec((LBLK, IN), lambda i: (i, 0)),
            pl.BlockSpec((IN, 4 * H), lambda i: (0, 0)),
            pl.BlockSpec((1, 4 * H), lambda i: (0, 0)),
            pl.BlockSpec((H, H), lambda i: (0, 0)),
            pl.BlockSpec((1, LBLK, 1), lambda i: (i, 0, 0)),
        ],
        out_specs=pl.BlockSpec((LBLK, H), lambda i: (i, 0)),
        out_shape=jax.ShapeDtypeStruct((TROWS, H), f32),
    )(xlp, wihT, bias, gcn1_w, dinv4)

    s1 = _sc_spmm(_sc_convert(ht1), srcb, dst2d, zeros_hbm)

    jblk = N // MBLK
    ht2 = pl.pallas_call(
        _combine_gcn2_body,
        grid=(B, jblk),
        in_specs=[
            pl.BlockSpec((1, MBLK, H), lambda b, j: (b, j, 0)),
            pl.BlockSpec((MBLK, H), lambda b, j: (b * jblk + j, 0)),
            pl.BlockSpec((1, MBLK, 1), lambda b, j: (j, 0, 0)),
            pl.BlockSpec((1, H), lambda b, j: (0, 0)),
            pl.BlockSpec((H, H), lambda b, j: (0, 0)),
        ],
        out_specs=pl.BlockSpec((MBLK, H), lambda b, j: (b * jblk + j, 0)),
        out_shape=jax.ShapeDtypeStruct((TROWS, H), f32),
    )(s1, ht1, dinv3d, gcn1_b.reshape(1, H), gcn2_w)

    s2 = _sc_spmm(_sc_convert(ht2), srcb, dst2d, zeros_hbm)

    out = pl.pallas_call(
        _final_body,
        grid=(B, jblk),
        in_specs=[
            pl.BlockSpec((1, MBLK, H), lambda b, j: (b, j, 0)),
            pl.BlockSpec((MBLK, H), lambda b, j: (b * jblk + j, 0)),
            pl.BlockSpec((1, MBLK, 1), lambda b, j: (j, 0, 0)),
            pl.BlockSpec((1, H), lambda b, j: (0, 0)),
            pl.BlockSpec((H, H // 2), lambda b, j: (0, 0)),
            pl.BlockSpec((1, H // 2), lambda b, j: (0, 0)),
            pl.BlockSpec((H // 2, OUT), lambda b, j: (0, 0)),
            pl.BlockSpec((1, OUT), lambda b, j: (0, 0)),
        ],
        out_specs=pl.BlockSpec((MBLK, OUT), lambda b, j: (b * jblk + j, 0)),
        out_shape=jax.ShapeDtypeStruct((B * N, OUT), f32),
    )(s2, ht2, dinv3d, gcn2_b.reshape(1, H), fc1_w, fc1_b.reshape(1, H // 2),
      fc2_w, fc2_b.reshape(1, OUT))

    return out.reshape(B, N, OUT)


# speedup vs baseline: 1.0379x; 1.0379x over previous
import jax
import jax.numpy as jnp
from jax import lax
from jax.experimental import pallas as pl
from jax.experimental.pallas import tpu as pltpu
from jax.experimental.pallas import tpu_sc as plsc

N = 10000
E = 320000
B = 4
IN = 128
H = 128
OUT = 16

NCORES = 2
NTILES = 16
BPC = B // NCORES
PADN = 10240
RPT = PADN // NTILES
CHUNK = 128
EPAD = 327680
NCHUNKS = EPAD // CHUNK
CPT = NCHUNKS // NTILES
CPT_DEG = NCHUNKS // (2 * NTILES)
NBUF = 4
MBLK = 400
TROWS = 40960
LBLK = 512
CVT_RPT = TROWS // 32
CVT_CH = 64
CVT_NCH = CVT_RPT // CVT_CH

_SC_MESH = dict(core_axis_name="c", subcore_axis_name="s")


def _lstm_gcn1_body(x_ref, wih_ref, bias_ref, w1_ref, dinv_ref, out_ref):
    xg = x_ref[...] @ wih_ref[...] + bias_ref[...]
    i_g = xg[:, 0:H]
    g_g = xg[:, 2 * H:3 * H]
    o_g = xg[:, 3 * H:4 * H]
    c = jax.nn.sigmoid(i_g) * jnp.tanh(g_g)
    h = jax.nn.sigmoid(o_g) * jnp.tanh(c)
    h1 = (h @ w1_ref[...]) * dinv_ref[0]
    out_ref[...] = h1


def _combine_gcn2_body(s_ref, ht_ref, dinv_ref, b1_ref, w2_ref, out_ref):
    dinv = dinv_ref[0]
    s = s_ref[0].astype(jnp.float32)
    ht = ht_ref[...].astype(jnp.float32)
    x2 = jax.nn.relu(dinv * (s + ht) + b1_ref[...])
    h2 = (x2 @ w2_ref[...]) * dinv
    out_ref[...] = h2


def _final_body(s_ref, ht_ref, dinv_ref, b2_ref, w3_ref, b3_ref, w4_ref,
                b4_ref, out_ref):
    dinv = dinv_ref[0]
    s = s_ref[0].astype(jnp.float32)
    ht = ht_ref[...].astype(jnp.float32)
    x3 = jax.nn.relu(dinv * (s + ht) + b2_ref[...])
    y = jax.nn.relu(x3 @ w3_ref[...] + b3_ref[...])
    out_ref[...] = y @ w4_ref[...] + b4_ref[...]


def _dinv_body(deg_ref, out_ref):
    out_ref[...] = lax.rsqrt(deg_ref[...] + 1.0)


def _deg_body(dst2d, out, didx_v, ones_v, acc_sh, sem):
    core = lax.axis_index("c")
    sub = lax.axis_index("s")
    zeros16 = jnp.zeros((16,), jnp.float32)
    ones16 = jnp.ones((16,), jnp.float32)
    for k in range(CHUNK // 16):
        ones_v[pl.ds(k * 16, 16)] = ones16

    def zrow(zv):
        for k in range(RPT // 16):
            zv[pl.ds(k * 16, 16)] = zeros16
        pltpu.sync_copy(zv, acc_sh.at[pl.ds(sub * RPT, RPT)])
    pl.run_scoped(zrow, pltpu.VMEM((RPT,), jnp.float32))
    plsc.subcore_barrier()
    base = (core * NTILES + sub) * CPT_DEG
    pltpu.sync_copy(dst2d.at[pl.ds(base, CPT_DEG)], didx_v)

    def body(j, _):
        pltpu.sync_copy(ones_v, acc_sh.at[didx_v.at[j]], add=True)
        return 0

    lax.fori_loop(0, CPT_DEG, body, 0)
    plsc.subcore_barrier()
    pltpu.sync_copy(acc_sh.at[pl.ds(sub * RPT, RPT)],
                    out.at[core, pl.ds(sub * RPT, RPT)])


def _cvt_body(tablef, tbl16, fbuf, bbuf):
    core = lax.axis_index("c")
    sub = lax.axis_index("s")
    gid = core * NTILES + sub
    r0 = gid * CVT_RPT
    iota2 = lax.iota(jnp.int32, 16) * 2
    idx_eo = [(iota2 + c * 32, iota2 + c * 32 + 1) for c in range(4)]

    def chunk(k, _):
        base = r0 + k * CVT_CH
        pltpu.sync_copy(tablef.at[pl.ds(base, CVT_CH)], fbuf)

        def row(r, _):
            ridx = jnp.full((16,), r, jnp.int32)
            for c in range(4):
                a = plsc.load_gather(fbuf, [ridx, idx_eo[c][0]])
                b = plsc.load_gather(fbuf, [ridx, idx_eo[c][1]])
                packed = plsc.pack(a, b, format=plsc.PackFormat.INTERLEAVED)
                bbuf[r, pl.ds(c * 32, 32)] = packed
            return 0

        lax.fori_loop(0, CVT_CH, row, 0)
        pltpu.sync_copy(bbuf, tbl16.at[pl.ds(base, CVT_CH)])
        return 0

    lax.fori_loop(0, CVT_NCH, chunk, 0)


def _sc_convert(tablef):
    kfn = pl.kernel(
        _cvt_body,
        out_type=jax.ShapeDtypeStruct((TROWS, H), jnp.bfloat16),
        mesh=plsc.VectorSubcoreMesh(**_SC_MESH),
        compiler_params=pltpu.CompilerParams(use_tc_tiling_on_sc=False,
                                             needs_layout_passes=False),
        scratch_types=[
            pltpu.VMEM((CVT_CH, H), jnp.float32),
            pltpu.VMEM((CVT_CH, H), jnp.bfloat16),
        ],
    )
    return kfn(tablef)


def _spmm_body(table, srcb, dst2d, zeros_hbm, out, sidx_v, didx_v, rows_v,
               frow_v, acc_sh, gsem, ssem):
    core = lax.axis_index("c")
    sub = lax.axis_index("s")
    row0 = sub * RPT
    cbase = sub * CPT
    iota2 = lax.iota(jnp.int32, 16) * 2
    idx_eo = [(iota2 + c * 32, iota2 + c * 32 + 1) for c in range(4)]
    pltpu.sync_copy(dst2d.at[pl.ds(cbase, CPT)], didx_v)

    for bl in range(BPC):
        batch = core * BPC + bl
        pltpu.sync_copy(zeros_hbm.at[pl.ds(row0, RPT)],
                        acc_sh.at[pl.ds(row0, RPT)])
        pltpu.sync_copy(srcb.at[batch, pl.ds(cbase, CPT)], sidx_v)
        plsc.subcore_barrier()
        for k in range(NBUF):
            pltpu.async_copy(table.at[sidx_v.at[k]], rows_v.at[k],
                             gsem.at[k])

        def body(j, _):
            s = lax.rem(j, NBUF)
            pltpu.make_async_copy(table.at[sidx_v.at[j]], rows_v.at[s],
                                  gsem.at[s]).wait()
            pltpu.async_copy(rows_v.at[s], acc_sh.at[didx_v.at[j]],
                             ssem.at[s], add=True)
            jn = j + NBUF

            @pl.when(jn < CPT)
            def _():
                pltpu.make_async_copy(rows_v.at[s], acc_sh.at[didx_v.at[j]],
                                      ssem.at[s]).wait()
                pltpu.async_copy(table.at[sidx_v.at[jn]], rows_v.at[s],
                                 gsem.at[s])

            return 0

        lax.fori_loop(0, CPT, body, 0)
        for k in range(NBUF):
            pltpu.make_async_copy(rows_v.at[k], acc_sh.at[didx_v.at[0]],
                                  ssem.at[k]).wait()
        plsc.subcore_barrier()
        for k in range(RPT // CHUNK):
            pltpu.sync_copy(acc_sh.at[pl.ds(row0 + k * CHUNK, CHUNK)],
                            rows_v.at[0])

            def row(r, _):
                ridx = jnp.full((16,), r, jnp.int32)
                for c in range(4):
                    ab = rows_v[0, r, pl.ds(c * 32, 32)]
                    a, b = plsc.unpack(ab, format=plsc.PackFormat.INTERLEAVED)
                    plsc.store_scatter(frow_v, [ridx, idx_eo[c][0]], a)
                    plsc.store_scatter(frow_v, [ridx, idx_eo[c][1]], b)
                return 0

            lax.fori_loop(0, CHUNK, row, 0)
            pltpu.sync_copy(frow_v,
                            out.at[batch, pl.ds(row0 + k * CHUNK, CHUNK)])


def _sc_degree(dst2d):
    kfn = pl.kernel(
        _deg_body,
        out_type=jax.ShapeDtypeStruct((NCORES, PADN), jnp.float32),
        mesh=plsc.VectorSubcoreMesh(**_SC_MESH),
        scratch_types=[
            pltpu.VMEM((CPT_DEG, CHUNK), jnp.int32),
            pltpu.VMEM((CHUNK,), jnp.float32),
            pltpu.VMEM_SHARED((PADN,), jnp.float32),
            pltpu.SemaphoreType.DMA,
        ],
    )
    return kfn(dst2d)


def _sc_spmm(table, srcb, dst2d, zeros_hbm):
    kfn = pl.kernel(
        _spmm_body,
        out_type=jax.ShapeDtypeStruct((B, PADN, H), jnp.float32),
        mesh=plsc.VectorSubcoreMesh(**_SC_MESH),
        compiler_params=pltpu.CompilerParams(use_tc_tiling_on_sc=False,
                                             needs_layout_passes=False),
        scratch_types=[
            pltpu.VMEM((CPT, CHUNK), jnp.int32),
            pltpu.VMEM((CPT, CHUNK), jnp.int32),
            pltpu.VMEM((NBUF, CHUNK, H), jnp.bfloat16),
            pltpu.VMEM((CHUNK, H), jnp.float32),
            pltpu.VMEM_SHARED((PADN, H), jnp.bfloat16),
            pltpu.SemaphoreType.DMA((NBUF,)),
            pltpu.SemaphoreType.DMA((NBUF,)),
        ],
    )
    return kfn(table, srcb, dst2d, zeros_hbm)


def kernel(x, edge_index, w_ih, w_hh, b_ih, b_hh, gcn1_w, gcn1_b, gcn2_w,
           gcn2_b, fc1_w, fc1_b, fc2_w, fc2_b):
    f32 = jnp.float32
    bf16 = jnp.bfloat16
    i32 = jnp.int32
    xl = x.reshape(B * N, IN)
    src = edge_index[0]
    dst = edge_index[1]

    pad = EPAD - E
    pad_src = (jnp.arange(pad, dtype=i32) % N)
    pad_dst = N + (jnp.arange(pad, dtype=i32) % (PADN - N))
    src2d = jnp.concatenate([src, pad_src]).reshape(NCHUNKS, CHUNK)
    dst2d = jnp.concatenate([dst, pad_dst]).reshape(NCHUNKS, CHUNK)
    srcb = src2d[None] + (jnp.arange(B, dtype=i32) * N)[:, None, None]
    zeros_hbm = jnp.zeros((PADN, H), bf16)

    deg_parts = _sc_degree(dst2d)
    dinv = pl.pallas_call(
        _dinv_body,
        out_shape=jax.ShapeDtypeStruct((PADN // 128, 128), f32),
    )(
        (deg_parts[0] + deg_parts[1]).reshape(PADN // 128, 128))
    dinv_n = dinv.reshape(PADN)[:N]
    dinv3d = dinv_n.reshape(N // MBLK, MBLK, 1)
    dinv4 = jnp.concatenate(
        [jnp.tile(dinv_n, B), jnp.ones((TROWS - B * N,), f32)]
    ).reshape(TROWS // LBLK, LBLK, 1)
    xlp = jnp.concatenate([xl, jnp.zeros((TROWS - B * N, IN), f32)])

    wihT = w_ih.T
    bias = (b_ih + b_hh).reshape(1, 4 * H)
    nblk = TROWS // LBLK
    ht1 = pl.pallas_call(
        _lstm_gcn1_body,
        grid=(nblk,),
        in_specs=[
            pl.BlockSpec((LBLK, IN), lambda i: (i, 0)),
            pl.BlockSpec((IN, 4 * H), lambda i: (0, 0)),
            pl.BlockSpec((1, 4 * H), lambda i: (0, 0)),
            pl.BlockSpec((H, H), lambda i: (0, 0)),
            pl.BlockSpec((1, LBLK, 1), lambda i: (i, 0, 0)),
        ],
        out_specs=pl.BlockSpec((LBLK, H), lambda i: (i, 0)),
        out_shape=jax.ShapeDtypeStruct((TROWS, H), f32),
    )(xlp, wihT, bias, gcn1_w, dinv4)

    s1 = _sc_spmm(_sc_convert(ht1), srcb, dst2d, zeros_hbm)

    jblk = N // MBLK
    ht2 = pl.pallas_call(
        _combine_gcn2_body,
        grid=(B, jblk),
        in_specs=[
            pl.BlockSpec((1, MBLK, H), lambda b, j: (b, j, 0)),
            pl.BlockSpec((MBLK, H), lambda b, j: (b * jblk + j, 0)),
            pl.BlockSpec((1, MBLK, 1), lambda b, j: (j, 0, 0)),
            pl.BlockSpec((1, H), lambda b, j: (0, 0)),
            pl.BlockSpec((H, H), lambda b, j: (0, 0)),
        ],
        out_specs=pl.BlockSpec((MBLK, H), lambda b, j: (b * jblk + j, 0)),
        out_shape=jax.ShapeDtypeStruct((TROWS, H), f32),
    )(s1, ht1, dinv3d, gcn1_b.reshape(1, H), gcn2_w)

    s2 = _sc_spmm(_sc_convert(ht2), srcb, dst2d, zeros_hbm)

    out = pl.pallas_call(
        _final_body,
        grid=(B, jblk),
        in_specs=[
            pl.BlockSpec((1, MBLK, H), lambda b, j: (b, j, 0)),
            pl.BlockSpec((MBLK, H), lambda b, j: (b * jblk + j, 0)),
            pl.BlockSpec((1, MBLK, 1), lambda b, j: (j, 0, 0)),
            pl.BlockSpec((1, H), lambda b, j: (0, 0)),
            pl.BlockSpec((H, H // 2), lambda b, j: (0, 0)),
            pl.BlockSpec((1, H // 2), lambda b, j: (0, 0)),
            pl.BlockSpec((H // 2, OUT), lambda b, j: (0, 0)),
            pl.BlockSpec((1, OUT), lambda b, j: (0, 0)),
        ],
        out_specs=pl.BlockSpec((MBLK, OUT), lambda b, j: (b * jblk + j, 0)),
        out_shape=jax.ShapeDtypeStruct((B * N, OUT), f32),
    )(s2, ht2, dinv3d, gcn2_b.reshape(1, H), fc1_w, fc1_b.reshape(1, H // 2),
      fc2_w, fc2_b.reshape(1, OUT))

    return out.reshape(B, N, OUT)
